# baseline (speedup 1.0000x reference)
import functools

import jax
import jax.numpy as jnp
from jax import lax
from jax.experimental import pallas as pl
from jax.experimental.pallas import tpu as pltpu
from jax.experimental.pallas import tpu_sc as plsc

_VOCAB2 = 1002
_NPOS = 24
_EMBED = 128
_NC, _NS = 2, 16
_NW = _NC * _NS
_BATCH = 1024
_HIST = 200
_N = _BATCH * _HIST
_BR_W = _BATCH // _NW
_NBUF = 4
_LOOK = 3
_NGRP = _BR_W // _NBUF
_SPLIT = 128
_REM = _HIST - _SPLIT


def _build_comb(W_word, W_pos):
    def body(w_ref, p_ref, comb_ref):
        comb_ref[...] = w_ref[...][:, None, :] + p_ref[...][None, :, :]

    comb = pl.pallas_call(
        body,
        grid=(2,),
        in_specs=[
            pl.BlockSpec((512, _EMBED), lambda i: (i, 0)),
            pl.BlockSpec((_NPOS, _EMBED), lambda i: (0, 0)),
        ],
        out_specs=pl.BlockSpec((512, _NPOS, _EMBED), lambda i: (i, 0, 0)),
        out_shape=jax.ShapeDtypeStruct((_VOCAB2, _NPOS, _EMBED), jnp.float32),
    )(W_word, W_pos)
    return comb.reshape(_VOCAB2 * _NPOS, _EMBED)


def _sc_lookup(cidxa, cidxb, wcomb):
    mesh = plsc.VectorSubcoreMesh(
        core_axis_name="c", subcore_axis_name="s",
        num_cores=_NC, num_subcores=_NS)

    @functools.partial(
        pl.kernel,
        out_type=jax.ShapeDtypeStruct((_N, _EMBED), jnp.float32),
        mesh=mesh,
        scratch_types=[
            pltpu.VMEM((_BR_W, _SPLIT), jnp.int32),
            pltpu.VMEM((_BR_W, _SPLIT), jnp.int32),
            [pltpu.VMEM((_HIST, _EMBED), jnp.float32) for _ in range(_NBUF)],
            [pltpu.SemaphoreType.DMA for _ in range(_NBUF)],
            [pltpu.SemaphoreType.DMA for _ in range(_NBUF)],
            [pltpu.SemaphoreType.DMA for _ in range(_NBUF)],
        ],
    )
    def k(cidxa_hbm, cidxb_hbm, comb_hbm, out_hbm, cidxa_v, cidxb_v,
          rows, gsemA, gsemB, ssem):
        c = lax.axis_index("c")
        s = lax.axis_index("s")
        wid = s * _NC + c
        rb = wid * _BR_W

        pltpu.sync_copy(cidxa_hbm.at[pl.ds(rb, _BR_W)], cidxa_v)
        pltpu.sync_copy(cidxb_hbm.at[pl.ds(rb, _BR_W)], cidxb_v)

        slA, slB = pl.ds(0, _SPLIT), pl.ds(_SPLIT, _REM)

        def start_gather(b, r):
            pltpu.async_copy(comb_hbm.at[cidxa_v.at[r]],
                             rows[b].at[slA], gsemA[b])
            pltpu.async_copy(comb_hbm.at[cidxb_v.at[r, pl.ds(0, _REM)]],
                             rows[b].at[slB], gsemB[b])

        def move(b, r):
            pltpu.make_async_copy(comb_hbm.at[cidxa_v.at[0]],
                                  rows[b].at[slA], gsemA[b]).wait()
            pltpu.async_copy(rows[b].at[slA],
                             out_hbm.at[pl.ds((rb + r) * _HIST, _SPLIT)],
                             ssem[b])
            pltpu.make_async_copy(comb_hbm.at[cidxb_v.at[0, pl.ds(0, _REM)]],
                                  rows[b].at[slB], gsemB[b]).wait()
            pltpu.async_copy(rows[b].at[slB],
                             out_hbm.at[pl.ds((rb + r) * _HIST + _SPLIT, _REM)],
                             ssem[b])

        def wait_scatter(b):
            pltpu.make_async_copy(rows[b].at[slA],
                                  out_hbm.at[pl.ds(0, _SPLIT)], ssem[b]).wait()
            pltpu.make_async_copy(rows[b].at[slB],
                                  out_hbm.at[pl.ds(0, _REM)], ssem[b]).wait()

        for b in range(_LOOK):
            start_gather(b, b)

        def group(g, carry):
            base = g * _NBUF
            for b in range(_NBUF):
                r = base + b
                bg = (b + _LOOK) % _NBUF

                @pl.when(r + _LOOK < _BR_W)
                def _():
                    @pl.when(r >= _NBUF - _LOOK)
                    def _():
                        wait_scatter(bg)
                    start_gather(bg, r + _LOOK)

                move(b, r)
            return carry
        lax.fori_loop(0, _NGRP, group, 0)

        for b in range(_NBUF):
            wait_scatter(b)

    return k(cidxa, cidxb, wcomb)


def kernel(tokens, pos, W_word, W_pos):
    wcomb = _build_comb(W_word, W_pos)
    tok = tokens.astype(jnp.int32)
    post = pos.astype(jnp.int32)
    cidxa = tok[:, :_SPLIT] * _NPOS + post[:, :_SPLIT]
    cidxb = jnp.pad(tok[:, _SPLIT:] * _NPOS + post[:, _SPLIT:],
                    ((0, 0), (0, 2 * _SPLIT - _HIST)))
    out = _sc_lookup(cidxa, cidxb, wcomb)
    return out.reshape(_BATCH, _HIST, _EMBED)


# speedup vs baseline: 1.0196x; 1.0196x over previous
import functools

import jax
import jax.numpy as jnp
from jax import lax
from jax.experimental import pallas as pl
from jax.experimental.pallas import tpu as pltpu
from jax.experimental.pallas import tpu_sc as plsc

_VOCAB2 = 1002
_NPOS = 24
_EMBED = 128
_NC, _NS = 2, 16
_NW = _NC * _NS
_BATCH = 1024
_HIST = 200
_N = _BATCH * _HIST
_BR_W = _BATCH // _NW
_NBUF = 4
_LOOK = 3
_NGRP = _BR_W // _NBUF
_SPLIT = 128
_REM = _HIST - _SPLIT


def _build_comb(W_word, W_pos):
    def body(w_ref, p_ref, comb_ref):
        comb_ref[...] = w_ref[...][:, None, :] + p_ref[...][None, :, :]

    comb = pl.pallas_call(
        body,
        grid=(2,),
        in_specs=[
            pl.BlockSpec((512, _EMBED), lambda i: (i, 0)),
            pl.BlockSpec((_NPOS, _EMBED), lambda i: (0, 0)),
        ],
        out_specs=pl.BlockSpec((512, _NPOS, _EMBED), lambda i: (i, 0, 0)),
        out_shape=jax.ShapeDtypeStruct((_VOCAB2, _NPOS, _EMBED), jnp.float32),
    )(W_word, W_pos)
    return comb.reshape(_VOCAB2 * _NPOS, _EMBED)


def _sc_lookup(cidxa, cidxb, wcomb):
    mesh = plsc.VectorSubcoreMesh(
        core_axis_name="c", subcore_axis_name="s",
        num_cores=_NC, num_subcores=_NS)

    @functools.partial(
        pl.kernel,
        out_type=jax.ShapeDtypeStruct((_N, _EMBED), jnp.float32),
        mesh=mesh,
        scratch_types=[
            pltpu.VMEM((_BR_W, _SPLIT), jnp.int32),
            pltpu.VMEM((_BR_W, _SPLIT), jnp.int32),
            [pltpu.VMEM((_HIST, _EMBED), jnp.float32) for _ in range(_NBUF)],
            [pltpu.SemaphoreType.DMA for _ in range(_NBUF)],
            [pltpu.SemaphoreType.DMA for _ in range(_NBUF)],
            [pltpu.SemaphoreType.DMA for _ in range(_NBUF)],
        ],
    )
    def k(cidxa_hbm, cidxb_hbm, comb_hbm, out_hbm, cidxa_v, cidxb_v,
          rows, gsemA, gsemB, ssem):
        c = lax.axis_index("c")
        s = lax.axis_index("s")
        wid = s * _NC + c
        rb = wid * _BR_W

        pltpu.sync_copy(cidxa_hbm.at[pl.ds(rb, _BR_W)], cidxa_v)
        pltpu.sync_copy(cidxb_hbm.at[pl.ds(rb, _BR_W)], cidxb_v)

        slA, slB = pl.ds(0, _SPLIT), pl.ds(_SPLIT, _REM)

        def start_gather(b, r):
            pltpu.async_copy(comb_hbm.at[cidxa_v.at[r]],
                             rows[b].at[slA], gsemA[b])
            pltpu.async_copy(comb_hbm.at[cidxb_v.at[r, pl.ds(0, _REM)]],
                             rows[b].at[slB], gsemB[b])

        def move(b, r):
            pltpu.make_async_copy(comb_hbm.at[cidxa_v.at[0]],
                                  rows[b].at[slA], gsemA[b]).wait()
            pltpu.async_copy(rows[b].at[slA],
                             out_hbm.at[pl.ds((rb + r) * _HIST, _SPLIT)],
                             ssem[b])
            pltpu.make_async_copy(comb_hbm.at[cidxb_v.at[0, pl.ds(0, _REM)]],
                                  rows[b].at[slB], gsemB[b]).wait()
            pltpu.async_copy(rows[b].at[slB],
                             out_hbm.at[pl.ds((rb + r) * _HIST + _SPLIT, _REM)],
                             ssem[b])

        def wait_scatter(b):
            pltpu.make_async_copy(rows[b].at[slA],
                                  out_hbm.at[pl.ds(0, _SPLIT)], ssem[b]).wait()
            pltpu.make_async_copy(rows[b].at[slB],
                                  out_hbm.at[pl.ds(0, _REM)], ssem[b]).wait()

        for b in range(_LOOK):
            start_gather(b, b)

        def group(g, carry):
            base = g * _NBUF
            for b in range(_NBUF):
                r = base + b
                bg = (b + _LOOK) % _NBUF

                @pl.when(r + _LOOK < _BR_W)
                def _():
                    @pl.when(r >= _NBUF - _LOOK)
                    def _():
                        wait_scatter(bg)
                    start_gather(bg, r + _LOOK)

                move(b, r)
            return carry
        lax.fori_loop(0, _NGRP, group, 0)

        for b in range(_NBUF):
            wait_scatter(b)

    return k(cidxa, cidxb, wcomb)


def kernel(tokens, pos, W_word, W_pos):
    wcomb = _build_comb(W_word, W_pos)
    cidx = tokens.astype(jnp.int32) * _NPOS + pos.astype(jnp.int32)
    cidxa = cidx[:, :_SPLIT]
    cidxb = jnp.pad(cidx[:, _SPLIT:], ((0, 0), (0, 2 * _SPLIT - _HIST)))
    out = _sc_lookup(cidxa, cidxb, wcomb)
    return out.reshape(_BATCH, _HIST, _EMBED)
